# baseline (speedup 1.0000x reference)
import functools

import jax
import jax.numpy as jnp
from jax import lax
from jax.experimental import pallas as pl
from jax.experimental.pallas import tpu as pltpu
from jax.experimental.pallas import tpu_sc as plsc

_SEQ = 1024
_D = 64
_NP = 128
_NK = 15
_PH_PER_W = _NP // 32
_ROWS_PER_PH = _SEQ // _NP


def _sc_toeplitz(s_tbl):
    mesh = plsc.VectorSubcoreMesh(core_axis_name="c", subcore_axis_name="s",
                                  num_cores=2)

    @functools.partial(
        pl.kernel,
        mesh=mesh,
        out_type=jax.ShapeDtypeStruct((_SEQ, 8, 8, 8, _NP), jnp.float32),
        scratch_types=[
            pltpu.VMEM((8, _NK, 8, _NP), jnp.float32),
            pltpu.SemaphoreType.DMA,
        ],
    )
    def k(s_hbm, out_hbm, slab, sem):
        c = lax.axis_index("c")
        s = lax.axis_index("s")
        wid = c * 16 + s
        for q in range(_PH_PER_W):
            p = wid * _PH_PER_W + q
            pltpu.sync_copy(s_hbm.at[p], slab)
            copies = []
            for m in range(_ROWS_PER_PH):
                i = _SEQ - 1 - _NP * m - p
                for dt in range(8):
                    copies.append(
                        pltpu.async_copy(
                            slab.at[dt, pl.ds(m, 8)],
                            out_hbm.at[i, dt],
                            sem,
                        )
                    )
            for cp in copies:
                cp.wait()

    return k(s_tbl)


def kernel(seq_len, relative_position_matrix):
    del seq_len
    fu = jnp.flip(relative_position_matrix, axis=0)[_SEQ:3 * _SEQ, :]
    kidx = (jnp.arange(_NP)[:, None, None]
            + _NP * jnp.arange(_NK)[None, :, None]
            + jnp.arange(_NP)[None, None, :])
    g = fu[kidx]
    s_tbl = g.reshape(_NP, _NK, _NP, 8, 8).transpose(0, 3, 1, 4, 2)
    out5 = _sc_toeplitz(s_tbl)
    return out5.transpose(0, 2, 4, 1, 3).reshape(_SEQ, _SEQ, _D)
